# baseline (speedup 1.0000x reference)
import functools

import jax
import jax.numpy as jnp
from jax import lax
from jax.experimental import pallas as pl
from jax.experimental.pallas import tpu as pltpu
from jax.experimental.pallas import tpu_sc as plsc

_VOCAB = 1000
_VPAD = 1024
_B = 1024
_T = 50
_TPAD = 56
_NW = 32
_BPW = _B // _NW


def _make_gather():
    mesh = plsc.VectorSubcoreMesh(core_axis_name="c", subcore_axis_name="s")

    @functools.partial(
        pl.kernel,
        out_type=jax.ShapeDtypeStruct((_B, _TPAD, _VPAD), jnp.float32),
        mesh=mesh,
        compiler_params=pltpu.CompilerParams(use_tc_tiling_on_sc=True),
        scratch_types=[
            pltpu.VMEM((_BPW, _TPAD), jnp.int32),
            pltpu.VMEM((_TPAD, _VPAD), jnp.float32),
            pltpu.VMEM((_TPAD, _VPAD), jnp.float32),
            pltpu.SemaphoreType.DMA,
            pltpu.SemaphoreType.DMA,
            pltpu.SemaphoreType.DMA,
            pltpu.SemaphoreType.DMA,
        ],
    )
    def embed_gather(idx_hbm, table_hbm, out_hbm, idx_v, buf0, buf1,
                     gsem0, gsem1, wsem0, wsem1):
        wid = lax.axis_index("s") * 2 + lax.axis_index("c")
        base = wid * _BPW
        pltpu.sync_copy(idx_hbm.at[pl.ds(base, _BPW)], idx_v)

        bufs = (buf0, buf1)
        gsems = (gsem0, gsem1)
        wsems = (wsem0, wsem1)

        def gather_start(c, b):
            pltpu.async_copy(table_hbm.at[idx_v.at[c]], bufs[b], gsems[b])

        def gather_wait(b):
            pltpu.make_async_copy(
                table_hbm.at[idx_v.at[0]], bufs[b], gsems[b]).wait()

        def write_start(c, b):
            pltpu.async_copy(bufs[b], out_hbm.at[base + c], wsems[b])

        def write_wait(b):
            pltpu.make_async_copy(bufs[b], out_hbm.at[base], wsems[b]).wait()

        gather_start(0, 0)

        gather_start(1, 1)
        gather_wait(0)
        write_start(0, 0)

        @pl.loop(1, _BPW - 1, step=2)
        def _pair(g):
            for b in (1, 0):
                c = g if b == 1 else g + 1
                write_wait(1 - b)
                gather_start(c + 1, 1 - b)
                gather_wait(b)
                write_start(c, b)

        write_wait(0)
        gather_wait(1)
        write_start(_BPW - 1, 1)
        write_wait(1)

    return embed_gather


def kernel(idx, token_embedding_table):
    table_pad = jnp.pad(token_embedding_table, ((0, 0), (0, _VPAD - _VOCAB)))
    idx_pad = jnp.pad(idx, ((0, 0), (0, _TPAD - _T)))
    raw = _make_gather()(idx_pad, table_pad)
    return raw[:, :_T, :_VOCAB]


# speedup vs baseline: 1.1622x; 1.1622x over previous
import functools

import jax
import jax.numpy as jnp
from jax import lax
from jax.experimental import pallas as pl
from jax.experimental.pallas import tpu as pltpu
from jax.experimental.pallas import tpu_sc as plsc

_VOCAB = 1000
_B = 1024
_T = 50
_NW = 32
_BPW = _B // _NW


def _make_gather():
    mesh = plsc.VectorSubcoreMesh(core_axis_name="c", subcore_axis_name="s")

    @functools.partial(
        pl.kernel,
        out_type=jax.ShapeDtypeStruct((_B, _T, _VOCAB), jnp.float32),
        mesh=mesh,
        compiler_params=pltpu.CompilerParams(use_tc_tiling_on_sc=False),
        scratch_types=[
            pltpu.VMEM((_BPW, _T), jnp.int32),
            pltpu.VMEM((_T, _VOCAB), jnp.float32),
            pltpu.VMEM((_T, _VOCAB), jnp.float32),
            pltpu.SemaphoreType.DMA,
            pltpu.SemaphoreType.DMA,
            pltpu.SemaphoreType.DMA,
            pltpu.SemaphoreType.DMA,
        ],
    )
    def embed_gather(idx_hbm, table_hbm, out_hbm, idx_v, buf0, buf1,
                     gsem0, gsem1, wsem0, wsem1):
        wid = lax.axis_index("s") * 2 + lax.axis_index("c")
        base = wid * _BPW
        pltpu.sync_copy(idx_hbm.at[pl.ds(base, _BPW)], idx_v)

        bufs = (buf0, buf1)
        gsems = (gsem0, gsem1)
        wsems = (wsem0, wsem1)

        def gather_start(c, b):
            pltpu.async_copy(table_hbm.at[idx_v.at[c]], bufs[b], gsems[b])

        def gather_wait(b):
            pltpu.make_async_copy(
                table_hbm.at[idx_v.at[0]], bufs[b], gsems[b]).wait()

        def write_start(c, b):
            pltpu.async_copy(bufs[b], out_hbm.at[base + c], wsems[b])

        def write_wait(b):
            pltpu.make_async_copy(bufs[b], out_hbm.at[base], wsems[b]).wait()

        gather_start(0, 0)

        gather_start(1, 1)
        gather_wait(0)
        write_start(0, 0)

        @pl.loop(1, _BPW - 1, step=2)
        def _pair(g):
            for b in (1, 0):
                c = g if b == 1 else g + 1
                write_wait(1 - b)
                gather_start(c + 1, 1 - b)
                gather_wait(b)
                write_start(c, b)

        write_wait(0)
        gather_wait(1)
        write_start(_BPW - 1, 1)
        write_wait(1)

    return embed_gather


def kernel(idx, token_embedding_table):
    return _make_gather()(idx, token_embedding_table)
